# Initial kernel scaffold; baseline (speedup 1.0000x reference)
import jax
import jax.numpy as jnp
from jax.experimental import pallas as pl


def kernel(input_ids, scores, green):
    raise NotImplementedError("write your pallas kernel here")


# speedup vs baseline: 2.7540x; 2.7540x over previous
import functools

import jax
import jax.numpy as jnp
from jax import lax
from jax.experimental import pallas as pl
from jax.experimental.pallas import tpu as pltpu
from jax.experimental.pallas import tpu_sc as plsc

V = 100000
B = 64
VPAD = 100096
SEG = 6256
NV = SEG // 16
LSEG = V - 15 * SEG
LNV = LSEG // 16
KBASE = 0x3F800000
MAXK = 0x3FFFFFFF
ND = 1024
BIG = 1 << 30
TOPP = 0.9


def _iota():
    return lax.iota(jnp.int32, 16)


def _sc_kernel(p, w):
    mesh = plsc.VectorSubcoreMesh(core_axis_name="c", subcore_axis_name="s")

    @functools.partial(
        pl.kernel,
        mesh=mesh,
        out_type=jax.ShapeDtypeStruct((B * V,), jnp.float32),
        scratch_types=[
            pltpu.VMEM((SEG,), jnp.int32),
            pltpu.VMEM((SEG,), jnp.float32),
            pltpu.VMEM((SEG,), jnp.float32),
            pltpu.VMEM((SEG,), jnp.int32),
            pltpu.VMEM((ND,), jnp.int32),
            pltpu.VMEM((ND,), jnp.int32),
            pltpu.VMEM((16 * ND,), jnp.int32),
            pltpu.VMEM((SEG,), jnp.float32),
            pltpu.VMEM((16,), jnp.float32),
            pltpu.VMEM((16,), jnp.int32),
            pltpu.VMEM_SHARED((VPAD,), jnp.int32),
            pltpu.VMEM_SHARED((VPAD,), jnp.int32),
            pltpu.VMEM_SHARED((16 * ND,), jnp.int32),
            pltpu.VMEM_SHARED((256,), jnp.float32),
            pltpu.VMEM_SHARED((256,), jnp.int32),
            pltpu.SemaphoreType.DMA,
        ],
        compiler_params=pltpu.CompilerParams(needs_layout_passes=False),
    )
    def kern(p_hbm, w_hbm, out_hbm, kbuf, fbuf, wbuf, posbuf, hist, ctr,
             gridv, fillbuf, patchv, patchi, bufA, bufB, histgrid, redf,
             redi, sem):
        t = lax.axis_index("s")
        cid = lax.axis_index("c")

        def fill_step(j, _):
            fillbuf[pl.ds(j * 16, 16)] = jnp.full((16,), 1e-5, jnp.float32)
            return 0

        lax.fori_loop(0, NV, fill_step, 0)

        def radix_pass(row, sh, src, dst):
            def zstep(j, _):
                hist[pl.ds(j * 16, 16)] = jnp.zeros((16,), jnp.int32)
                return 0

            lax.fori_loop(0, ND // 16, zstep, 0)

            if src is None:
                off = row * V + t * SEG

                @pl.when(t < 15)
                def _():
                    pltpu.sync_copy(p_hbm.at[pl.ds(off, SEG)], fbuf)

                @pl.when(t == 15)
                def _():
                    pltpu.sync_copy(p_hbm.at[pl.ds(off, LSEG)],
                                    fbuf.at[pl.ds(0, LSEG)])

                rv = jnp.where(t == 15, LNV, NV)

                def cstep(j, _):
                    pv = fbuf[pl.ds(j * 16, 16)]
                    kv = KBASE - plsc.bitcast(pv, jnp.int32)
                    kv = jnp.where(j < rv, kv, jnp.int32(MAXK))
                    kbuf[pl.ds(j * 16, 16)] = kv
                    return 0

                lax.fori_loop(0, NV, cstep, 0)
            else:
                pltpu.sync_copy(src.at[pl.ds(t * SEG, SEG)], kbuf)

            def hstep(j, _):
                kv = kbuf[pl.ds(j * 16, 16)]
                d = (kv >> sh) & (ND - 1)
                occ, last = plsc.scan_count(d)
                h = plsc.load_gather(hist, [d])
                plsc.store_scatter(hist, [d], h + occ, mask=last)
                return 0

            lax.fori_loop(0, NV, hstep, 0)
            pltpu.sync_copy(hist, histgrid.at[pl.ds(t * ND, ND)])
            plsc.subcore_barrier()

            pltpu.sync_copy(histgrid, gridv)

            def sstep(dv, run):
                tot = jnp.zeros((16,), jnp.int32)
                mine = jnp.zeros((16,), jnp.int32)
                for tt in range(16):
                    g = gridv[pl.ds(tt * ND + dv * 16, 16)]
                    tot = tot + g
                    mine = mine + jnp.where(tt < t, g, jnp.int32(0))
                excl = run + (plsc.cumsum(tot) - tot)
                ctr[pl.ds(dv * 16, 16)] = excl + mine
                return run + jnp.sum(tot)

            lax.fori_loop(0, ND // 16, sstep, jnp.int32(0))

            def rstep(j, _):
                kv = kbuf[pl.ds(j * 16, 16)]
                d = (kv >> sh) & (ND - 1)
                occ, last = plsc.scan_count(d)
                c0 = plsc.load_gather(ctr, [d])
                posbuf[pl.ds(j * 16, 16)] = c0 + occ - 1
                plsc.store_scatter(ctr, [d], c0 + occ, mask=last)
                return 0

            lax.fori_loop(0, NV, rstep, 0)
            cp = pltpu.make_async_copy(kbuf, dst.at[posbuf], sem)
            cp.start()
            cp.wait()
            plsc.subcore_barrier()

        def do_row(rl, _):
            row = cid * 32 + rl
            radix_pass(row, 0, None, bufA)
            radix_pass(row, 10, bufA, bufB)
            radix_pass(row, 20, bufB, bufA)

            pltpu.sync_copy(bufA.at[pl.ds(t * SEG, SEG)], kbuf)

            def s0step(j, acc):
                kv = kbuf[pl.ds(j * 16, 16)]
                pv = plsc.bitcast(KBASE - kv, jnp.float32)
                return acc + jnp.where(pv > 0.0, pv, jnp.float32(0.0))

            acc = lax.fori_loop(0, NV, s0step, jnp.zeros((16,), jnp.float32))
            patchv[...] = jnp.full((16,), 1.0, jnp.float32) * jnp.sum(acc)
            pltpu.sync_copy(patchv, redf.at[pl.ds(t * 16, 16)])
            plsc.subcore_barrier()

            pltpu.sync_copy(redf, wbuf.at[pl.ds(0, 256)])
            segs = plsc.load_gather(wbuf, [_iota() * 16])
            base = jnp.sum(jnp.where(_iota() < t, segs, jnp.float32(0.0)))

            @pl.when(base < TOPP)
            def _():
                woff = row * V + t * SEG

                @pl.when(t < 15)
                def _():
                    pltpu.sync_copy(w_hbm.at[pl.ds(woff, SEG)], wbuf)

                @pl.when(t == 15)
                def _():
                    pltpu.sync_copy(w_hbm.at[pl.ds(woff, LSEG)],
                                    wbuf.at[pl.ds(0, LSEG)])

            def s1cond(c):
                j, run, bv, bi, bpv = c
                return (j < NV) & (run < TOPP)

            def s1body(c):
                j, run, bv, bi, bpv = c
                kv = kbuf[pl.ds(j * 16, 16)]
                pv = plsc.bitcast(KBASE - kv, jnp.float32)
                wv = wbuf[pl.ds(j * 16, 16)]
                cs = plsc.cumsum(pv)
                ecum = run + (cs - pv)
                keep = (ecum < TOPP) & (pv > 0.0)
                tv = jnp.where(keep, pv * wv, jnp.float32(-1.0))
                grank = t * SEG + j * 16 + _iota()
                better = tv > bv
                bv = jnp.where(better, tv, bv)
                bi = jnp.where(better, grank, bi)
                bpv = jnp.where(better, pv, bpv)
                pvp = jnp.where(pv > 0.0, pv, jnp.float32(0.0))
                return (j + 1, run + jnp.sum(pvp), bv, bi, bpv)

            _, _, bv, bi, bpv = lax.while_loop(
                s1cond, s1body,
                (jnp.int32(0), base,
                 jnp.full((16,), -2.0, jnp.float32),
                 jnp.full((16,), BIG, jnp.int32),
                 jnp.zeros((16,), jnp.float32)))

            m = jnp.max(bv)
            crank = jnp.min(jnp.where(bv == m, bi, jnp.int32(BIG)))
            cprob = jnp.max(jnp.where(bi == crank, bpv, jnp.float32(-1.0)))
            patchv[...] = jnp.where(
                _iota() == 0, m, jnp.where(_iota() == 1, cprob,
                                           jnp.float32(0.0)))
            pltpu.sync_copy(patchv, redf.at[pl.ds(t * 16, 16)])
            patchi[...] = jnp.full((16,), 1, jnp.int32) * crank
            pltpu.sync_copy(patchi, redi.at[pl.ds(t * 16, 16)])
            plsc.subcore_barrier()

            pltpu.sync_copy(redf, wbuf.at[pl.ds(0, 256)])
            pltpu.sync_copy(redi, gridv.at[pl.ds(0, 256)])
            mv = plsc.load_gather(wbuf, [_iota() * 16])
            pvv = plsc.load_gather(wbuf, [_iota() * 16 + 1])
            rnkv = plsc.load_gather(gridv, [_iota() * 16])
            mm = jnp.max(mv)
            istar = jnp.min(jnp.where(mv == mm, rnkv, jnp.int32(BIG)))
            vstar = jnp.max(jnp.where(rnkv == istar, pvv, jnp.float32(-1.0)))

            poff = row * V + t * SEG

            @pl.when(t < 15)
            def _():
                pltpu.sync_copy(p_hbm.at[pl.ds(poff, SEG)], fbuf)

            @pl.when(t == 15)
            def _():
                pltpu.sync_copy(p_hbm.at[pl.ds(poff, LSEG)],
                                fbuf.at[pl.ds(0, LSEG)])

            trips = jnp.where(t == 15, LNV, NV)

            def s3step(j, c):
                cgt, ceq, fj = c
                pv = fbuf[pl.ds(j * 16, 16)]
                gt = (pv > vstar).astype(jnp.int32)
                eq = pv == vstar
                gj = t * SEG + j * 16 + _iota()
                cand = jnp.min(jnp.where(eq, gj, jnp.int32(BIG)))
                return (cgt + jnp.sum(gt),
                        ceq + jnp.sum(eq.astype(jnp.int32)),
                        jnp.minimum(fj, cand))

            cgt, ceq, fj = lax.fori_loop(
                0, trips, s3step,
                (jnp.int32(0), jnp.int32(0), jnp.int32(BIG)))
            patchi[...] = jnp.where(
                _iota() == 0, cgt, jnp.where(_iota() == 1, ceq,
                                             jnp.where(_iota() == 2, fj,
                                                       jnp.int32(0))))
            pltpu.sync_copy(patchi, redi.at[pl.ds(t * 16, 16)])
            plsc.subcore_barrier()

            pltpu.sync_copy(redi, gridv.at[pl.ds(0, 256)])
            cgv = plsc.load_gather(gridv, [_iota() * 16])
            cev = plsc.load_gather(gridv, [_iota() * 16 + 1])
            fjv = plsc.load_gather(gridv, [_iota() * 16 + 2])
            cgt_tot = jnp.sum(cgv)
            cet_tot = jnp.sum(cev)
            mbase = jnp.sum(jnp.where(_iota() < t, cev, jnp.int32(0)))
            kth = istar - cgt_tot
            own = (mbase <= kth) & (kth < mbase + ceq)
            trip2 = jnp.where((cet_tot > 1) & own, trips, jnp.int32(0))

            def s4step(j, c):
                mc, js = c
                pv = fbuf[pl.ds(j * 16, 16)]
                eq = pv == vstar
                eqi = eq.astype(jnp.int32)
                glob = mc + plsc.cumsum(eqi)
                hit = eq & (glob == kth + 1)
                gj = t * SEG + j * 16 + _iota()
                cand = jnp.min(jnp.where(hit, gj, jnp.int32(BIG)))
                return (mc + jnp.sum(eqi), jnp.minimum(js, cand))

            _, js = lax.fori_loop(0, trip2, s4step, (mbase, jnp.int32(BIG)))
            jstar = jnp.where(cet_tot == 1, jnp.min(fjv), js)

            @pl.when(t < 15)
            def _():
                pltpu.sync_copy(fillbuf, out_hbm.at[pl.ds(poff, SEG)])

            @pl.when(t == 15)
            def _():
                pltpu.sync_copy(fillbuf.at[pl.ds(0, LSEG)],
                                out_hbm.at[pl.ds(poff, LSEG)])

            seg_end = t * SEG + trips * 16
            do_patch = (jstar < BIG) & (t * SEG <= jstar) & (jstar < seg_end)

            @pl.when(do_patch)
            def _():
                vb = (jstar // 16) * 16
                patchv[...] = jnp.where(_iota() == jstar - vb,
                                        jnp.float32(1e5), jnp.float32(1e-5))
                pltpu.sync_copy(patchv, out_hbm.at[pl.ds(row * V + vb, 16)])

            return 0

        lax.fori_loop(0, 32, do_row, 0)

    return kern(p, w)


def kernel(input_ids, scores, green):
    del input_ids
    logits = scores + 2.0 * green
    p = jax.nn.softmax(logits, axis=-1)
    keys = jax.random.split(jax.random.key(42), B)
    g = jax.vmap(lambda k_: jax.random.gumbel(k_, (V,), jnp.float32))(keys)
    w = jnp.exp(g)
    out = _sc_kernel(p.reshape(-1), w.reshape(-1))
    return out.reshape(B, V)


# speedup vs baseline: 2.8183x; 1.0233x over previous
import functools

import jax
import jax.numpy as jnp
from jax import lax
from jax.experimental import pallas as pl
from jax.experimental.pallas import tpu as pltpu
from jax.experimental.pallas import tpu_sc as plsc

V = 100000
B = 64
SEG = 6272
VPAD = 16 * SEG
NV = SEG // 16
LSEG = V - 15 * SEG
LNV = LSEG // 16
KBASE = 0x3F800000
MAXK = 0x3FFFFFFF
ND = 1024
NH = 8
BIG = 1 << 30
TOPP = 0.9


def _iota():
    return lax.iota(jnp.int32, 16)


def _sc_kernel(p, w):
    mesh = plsc.VectorSubcoreMesh(core_axis_name="c", subcore_axis_name="s")

    @functools.partial(
        pl.kernel,
        mesh=mesh,
        out_type=jax.ShapeDtypeStruct((B * V,), jnp.float32),
        scratch_types=[
            pltpu.VMEM((SEG,), jnp.int32),
            pltpu.VMEM((SEG,), jnp.float32),
            pltpu.VMEM((SEG,), jnp.float32),
            pltpu.VMEM((SEG,), jnp.int32),
            pltpu.VMEM((NH * ND,), jnp.int32),
            pltpu.VMEM((ND,), jnp.int32),
            pltpu.VMEM((16 * ND,), jnp.int32),
            pltpu.VMEM((SEG,), jnp.float32),
            pltpu.VMEM((16,), jnp.float32),
            pltpu.VMEM((16,), jnp.int32),
            pltpu.VMEM_SHARED((VPAD,), jnp.int32),
            pltpu.VMEM_SHARED((VPAD,), jnp.int32),
            pltpu.VMEM_SHARED((16 * ND,), jnp.int32),
            pltpu.VMEM_SHARED((256,), jnp.float32),
            pltpu.VMEM_SHARED((256,), jnp.int32),
            pltpu.SemaphoreType.DMA,
        ],
        compiler_params=pltpu.CompilerParams(needs_layout_passes=False),
    )
    def kern(p_hbm, w_hbm, out_hbm, kbuf, fbuf, wbuf, posbuf, hists, ctr,
             gridv, fillbuf, patchv, patchi, bufA, bufB, histgrid, redf,
             redi, sem):
        t = lax.axis_index("s")
        cid = lax.axis_index("c")

        def fill_step(j, _):
            fillbuf[pl.ds(j * 16, 16)] = jnp.full((16,), 1e-5, jnp.float32)
            return 0

        lax.fori_loop(0, NV, fill_step, 0)

        def radix_pass(row, sh, src, dst):
            def zstep(j, _):
                hists[pl.ds(j * 16, 16)] = jnp.zeros((16,), jnp.int32)
                return 0

            lax.fori_loop(0, NH * ND // 16, zstep, 0)

            if src is None:
                off = row * V + t * SEG

                @pl.when(t < 15)
                def _():
                    pltpu.sync_copy(p_hbm.at[pl.ds(off, SEG)], fbuf)

                @pl.when(t == 15)
                def _():
                    pltpu.sync_copy(p_hbm.at[pl.ds(off, LSEG)],
                                    fbuf.at[pl.ds(0, LSEG)])

                rv = jnp.where(t == 15, LNV, NV)
            else:
                pltpu.sync_copy(src.at[pl.ds(t * SEG, SEG)], kbuf)
                rv = None

            def hstep(jj, _):
                for u in range(NH):
                    j = jj * NH + u
                    if src is None:
                        pv = fbuf[pl.ds(j * 16, 16)]
                        kv = KBASE - plsc.bitcast(pv, jnp.int32)
                        kv = jnp.where(j < rv, kv, jnp.int32(MAXK))
                        kbuf[pl.ds(j * 16, 16)] = kv
                    else:
                        kv = kbuf[pl.ds(j * 16, 16)]
                    d = ((kv >> sh) & (ND - 1)) + u * ND
                    occ, last = plsc.scan_count(d)
                    h = plsc.load_gather(hists, [d])
                    plsc.store_scatter(hists, [d], h + occ, mask=last)
                return 0

            lax.fori_loop(0, NV // NH, hstep, 0)

            def mstep(j, _):
                acc = hists[pl.ds(j * 16, 16)]
                for u in range(1, NH):
                    acc = acc + hists[pl.ds(u * ND + j * 16, 16)]
                hists[pl.ds(j * 16, 16)] = acc
                return 0

            lax.fori_loop(0, ND // 16, mstep, 0)
            pltpu.sync_copy(hists.at[pl.ds(0, ND)],
                            histgrid.at[pl.ds(t * ND, ND)])
            plsc.subcore_barrier()

            pltpu.sync_copy(histgrid, gridv)

            def sstep(dv, run):
                tot = jnp.zeros((16,), jnp.int32)
                mine = jnp.zeros((16,), jnp.int32)
                for tt in range(16):
                    g = gridv[pl.ds(tt * ND + dv * 16, 16)]
                    tot = tot + g
                    mine = mine + jnp.where(tt < t, g, jnp.int32(0))
                excl = run + (plsc.cumsum(tot) - tot)
                ctr[pl.ds(dv * 16, 16)] = excl + mine
                return run + jnp.sum(tot)

            lax.fori_loop(0, ND // 16, sstep, jnp.int32(0))

            def rstep(jj, _):
                for u in range(NH):
                    j = jj * NH + u
                    kv = kbuf[pl.ds(j * 16, 16)]
                    d = (kv >> sh) & (ND - 1)
                    occ, last = plsc.scan_count(d)
                    c0 = plsc.load_gather(ctr, [d])
                    posbuf[pl.ds(j * 16, 16)] = c0 + occ - 1
                    plsc.store_scatter(ctr, [d], c0 + occ, mask=last)
                return 0

            lax.fori_loop(0, NV // NH, rstep, 0)
            cp = pltpu.make_async_copy(kbuf, dst.at[posbuf], sem)
            cp.start()
            cp.wait()
            plsc.subcore_barrier()

        def do_row(rl, _):
            row = cid * 32 + rl
            radix_pass(row, 0, None, bufA)
            radix_pass(row, 10, bufA, bufB)
            radix_pass(row, 20, bufB, bufA)

            pltpu.sync_copy(bufA.at[pl.ds(t * SEG, SEG)], kbuf)

            def s0step(jj, acc):
                for u in range(NH):
                    j = jj * NH + u
                    kv = kbuf[pl.ds(j * 16, 16)]
                    pv = plsc.bitcast(KBASE - kv, jnp.float32)
                    acc = acc + jnp.where(pv > 0.0, pv, jnp.float32(0.0))
                return acc

            acc = lax.fori_loop(0, NV // NH, s0step,
                                jnp.zeros((16,), jnp.float32))
            patchv[...] = jnp.full((16,), 1.0, jnp.float32) * jnp.sum(acc)
            pltpu.sync_copy(patchv, redf.at[pl.ds(t * 16, 16)])
            plsc.subcore_barrier()

            pltpu.sync_copy(redf, wbuf.at[pl.ds(0, 256)])
            segs = plsc.load_gather(wbuf, [_iota() * 16])
            base = jnp.sum(jnp.where(_iota() < t, segs, jnp.float32(0.0)))

            @pl.when(base < TOPP)
            def _():
                woff = row * V + t * SEG

                @pl.when(t < 15)
                def _():
                    pltpu.sync_copy(w_hbm.at[pl.ds(woff, SEG)], wbuf)

                @pl.when(t == 15)
                def _():
                    pltpu.sync_copy(w_hbm.at[pl.ds(woff, LSEG)],
                                    wbuf.at[pl.ds(0, LSEG)])

            UW = 4

            def s1cond(c):
                j, run, bv, bi, bpv = c
                return (j < NV // UW) & (run < TOPP)

            def s1body(c):
                j, run, bv, bi, bpv = c
                for u in range(UW):
                    jv = j * UW + u
                    kv = kbuf[pl.ds(jv * 16, 16)]
                    pv = plsc.bitcast(KBASE - kv, jnp.float32)
                    wv = wbuf[pl.ds(jv * 16, 16)]
                    cs = plsc.cumsum(pv)
                    ecum = run + (cs - pv)
                    keep = (ecum < TOPP) & (pv > 0.0)
                    tv = jnp.where(keep, pv * wv, jnp.float32(-1.0))
                    grank = t * SEG + jv * 16 + _iota()
                    better = tv > bv
                    bv = jnp.where(better, tv, bv)
                    bi = jnp.where(better, grank, bi)
                    bpv = jnp.where(better, pv, bpv)
                    pvp = jnp.where(pv > 0.0, pv, jnp.float32(0.0))
                    run = run + jnp.sum(pvp)
                return (j + 1, run, bv, bi, bpv)

            _, _, bv, bi, bpv = lax.while_loop(
                s1cond, s1body,
                (jnp.int32(0), base,
                 jnp.full((16,), -2.0, jnp.float32),
                 jnp.full((16,), BIG, jnp.int32),
                 jnp.zeros((16,), jnp.float32)))

            m = jnp.max(bv)
            crank = jnp.min(jnp.where(bv == m, bi, jnp.int32(BIG)))
            cprob = jnp.max(jnp.where(bi == crank, bpv, jnp.float32(-1.0)))
            patchv[...] = jnp.where(
                _iota() == 0, m, jnp.where(_iota() == 1, cprob,
                                           jnp.float32(0.0)))
            pltpu.sync_copy(patchv, redf.at[pl.ds(t * 16, 16)])
            patchi[...] = jnp.full((16,), 1, jnp.int32) * crank
            pltpu.sync_copy(patchi, redi.at[pl.ds(t * 16, 16)])
            plsc.subcore_barrier()

            pltpu.sync_copy(redf, wbuf.at[pl.ds(0, 256)])
            pltpu.sync_copy(redi, gridv.at[pl.ds(0, 256)])
            mv = plsc.load_gather(wbuf, [_iota() * 16])
            pvv = plsc.load_gather(wbuf, [_iota() * 16 + 1])
            rnkv = plsc.load_gather(gridv, [_iota() * 16])
            mm = jnp.max(mv)
            istar = jnp.min(jnp.where(mv == mm, rnkv, jnp.int32(BIG)))
            vstar = jnp.max(jnp.where(rnkv == istar, pvv, jnp.float32(-1.0)))

            poff = row * V + t * SEG

            @pl.when(t < 15)
            def _():
                pltpu.sync_copy(p_hbm.at[pl.ds(poff, SEG)], fbuf)

            @pl.when(t == 15)
            def _():
                pltpu.sync_copy(p_hbm.at[pl.ds(poff, LSEG)],
                                fbuf.at[pl.ds(0, LSEG)])

            def s3step(jj, c):
                cgt, ceq, fj = c
                for u in range(NH):
                    j = jj * NH + u
                    pv = fbuf[pl.ds(j * 16, 16)]
                    gj = t * SEG + j * 16 + _iota()
                    valid = gj < V
                    gt = ((pv > vstar) & valid).astype(jnp.int32)
                    eq = (pv == vstar) & valid
                    cand = jnp.min(jnp.where(eq, gj, jnp.int32(BIG)))
                    cgt = cgt + jnp.sum(gt)
                    ceq = ceq + jnp.sum(eq.astype(jnp.int32))
                    fj = jnp.minimum(fj, cand)
                return (cgt, ceq, fj)

            cgt, ceq, fj = lax.fori_loop(
                0, NV // NH, s3step,
                (jnp.int32(0), jnp.int32(0), jnp.int32(BIG)))
            patchi[...] = jnp.where(
                _iota() == 0, cgt, jnp.where(_iota() == 1, ceq,
                                             jnp.where(_iota() == 2, fj,
                                                       jnp.int32(0))))
            pltpu.sync_copy(patchi, redi.at[pl.ds(t * 16, 16)])
            plsc.subcore_barrier()

            pltpu.sync_copy(redi, gridv.at[pl.ds(0, 256)])
            cgv = plsc.load_gather(gridv, [_iota() * 16])
            cev = plsc.load_gather(gridv, [_iota() * 16 + 1])
            fjv = plsc.load_gather(gridv, [_iota() * 16 + 2])
            cgt_tot = jnp.sum(cgv)
            cet_tot = jnp.sum(cev)
            mbase = jnp.sum(jnp.where(_iota() < t, cev, jnp.int32(0)))
            kth = istar - cgt_tot
            own = (mbase <= kth) & (kth < mbase + ceq)
            trip2 = jnp.where((cet_tot > 1) & own, NV, jnp.int32(0))

            def s4step(j, c):
                mc, js = c
                pv = fbuf[pl.ds(j * 16, 16)]
                gj = t * SEG + j * 16 + _iota()
                eq = (pv == vstar) & (gj < V)
                eqi = eq.astype(jnp.int32)
                glob = mc + plsc.cumsum(eqi)
                hit = eq & (glob == kth + 1)
                cand = jnp.min(jnp.where(hit, gj, jnp.int32(BIG)))
                return (mc + jnp.sum(eqi), jnp.minimum(js, cand))

            _, js = lax.fori_loop(0, trip2, s4step, (mbase, jnp.int32(BIG)))
            jstar = jnp.where(cet_tot == 1, jnp.min(fjv), js)

            @pl.when(t < 15)
            def _():
                pltpu.sync_copy(fillbuf, out_hbm.at[pl.ds(poff, SEG)])

            @pl.when(t == 15)
            def _():
                pltpu.sync_copy(fillbuf.at[pl.ds(0, LSEG)],
                                out_hbm.at[pl.ds(poff, LSEG)])

            seg_end = jnp.minimum(t * SEG + SEG, V)
            do_patch = (jstar < BIG) & (t * SEG <= jstar) & (jstar < seg_end)

            @pl.when(do_patch)
            def _():
                vb = (jstar // 16) * 16
                patchv[...] = jnp.where(_iota() == jstar - vb,
                                        jnp.float32(1e5), jnp.float32(1e-5))
                pltpu.sync_copy(patchv, out_hbm.at[pl.ds(row * V + vb, 16)])

            return 0

        lax.fori_loop(0, 32, do_row, 0)

    return kern(p, w)


def kernel(input_ids, scores, green):
    del input_ids
    logits = scores + 2.0 * green
    p = jax.nn.softmax(logits, axis=-1)
    keys = jax.random.split(jax.random.key(42), B)
    g = jax.vmap(lambda k_: jax.random.gumbel(k_, (V,), jnp.float32))(keys)
    w = jnp.exp(g)
    out = _sc_kernel(p.reshape(-1), w.reshape(-1))
    return out.reshape(B, V)
